# baseline (speedup 1.0000x reference)
import functools

import jax
import jax.numpy as jnp
from jax import lax
from jax.experimental import pallas as pl
from jax.experimental.pallas import tpu as pltpu
from jax.experimental.pallas import tpu_sc as plsc

NUM_ACTIONS = 100000
EMBED_DIM = 32
BATCH = 16384
EPS = 1e-5

NC = 2
NS = 16
L = 16
NW = NC * NS
B_PER_W = BATCH // NW
GATHER_CHUNK = 128
N_CHUNKS = B_PER_W // GATHER_CHUNK
GROUPS_PER_CHUNK = GATHER_CHUNK // L

TCOLS = NUM_ACTIONS // 128
TAIL_ROWS = NUM_ACTIONS - TCOLS * 128
ROWS_PAD = NUM_ACTIONS + 96
PITCH = EMBED_DIM + 1
FLAT_PAD = ROWS_PAD * PITCH

_GATHER_DNUMS = lax.GatherDimensionNumbers(
    offset_dims=(), collapsed_slice_dims=(0,), start_index_map=(0,))


def _lane_broadcast(vec, j):
    sel = jnp.full((L, 1), j, jnp.int32)
    return lax.gather(vec, sel, _GATHER_DNUMS, slice_sizes=(1,),
                      mode=lax.GatherScatterMode.PROMISE_IN_BOUNDS)


def _rsqrt(x):
    xi = plsc.bitcast(x, jnp.int32)
    y = plsc.bitcast(jnp.int32(0x5F3759DF) - (xi >> 1), jnp.float32)
    half = x * 0.5
    y = y * (1.5 - half * y * y)
    y = y * (1.5 - half * y * y)
    return y


M = 8
PASSES = 3
MAIN_COLS = NW * M * PASSES
EXTRA_COLS = TCOLS - MAIN_COLS
COL_FLOATS = 128 * PITCH


@functools.partial(
    pl.kernel,
    out_type=jax.ShapeDtypeStruct((FLAT_PAD,), jnp.float32),
    mesh=plsc.VectorSubcoreMesh(core_axis_name="c", subcore_axis_name="s"),
    compiler_params=pltpu.CompilerParams(
        needs_layout_passes=False, use_tc_tiling_on_sc=True),
    scratch_types=[
        pltpu.VMEM((EMBED_DIM, M * 128), jnp.float32),
        pltpu.VMEM((EMBED_DIM, M * 128), jnp.float32),
        pltpu.VMEM((M * COL_FLOATS,), jnp.float32),
        pltpu.SemaphoreType.DMA,
        pltpu.SemaphoreType.DMA,
        pltpu.SemaphoreType.DMA,
    ],
)
def _sc_detile(table_t, tail_pad, out_hbm, buf_a, buf_b, row_buf,
               in_sem_a, in_sem_b, out_sem):
    wid = lax.axis_index("s") * NC + lax.axis_index("c")
    j0 = wid * (M * PASSES)

    iota_p = lax.iota(jnp.int32, L) * PITCH
    bufs = [buf_a, buf_b]
    in_sems = [in_sem_a, in_sem_b]

    def fire_in(p, buf, sem):
        return [
            pltpu.async_copy(
                table_t.at[pl.ds(8 * i, 8),
                           pl.ds((j0 + M * p) * 128, M * 128)],
                buf.at[pl.ds(8 * i, 8)], sem)
            for i in range(4)
        ]

    def transpose_col(buf, c2, base):
        for g in range(8):
            vs = [buf[c, pl.ds(c2 * 128 + g * L, L)]
                  for c in range(EMBED_DIM)]
            for c in range(EMBED_DIM):
                plsc.store_scatter(
                    row_buf, [iota_p + (base + g * L * PITCH + c)],
                    vs[c])

    in_flight = fire_in(0, bufs[0], in_sems[0])
    out_copy = None
    for p in range(PASSES):
        for cp in in_flight:
            cp.wait()
        if p + 1 < PASSES:
            in_flight = fire_in(p + 1, bufs[(p + 1) % 2], in_sems[(p + 1) % 2])
        if out_copy is not None:
            out_copy.wait()
        buf = bufs[p % 2]

        def col_body(c2, carry, buf=buf):
            transpose_col(buf, c2, c2 * COL_FLOATS)
            return carry
        lax.fori_loop(0, M, col_body, 0)
        out_copy = pltpu.async_copy(
            row_buf, out_hbm.at[pl.ds((j0 + M * p) * COL_FLOATS,
                                      M * COL_FLOATS)], out_sem)
    out_copy.wait()

    @pl.when(wid < EXTRA_COLS)
    def _extra():
        j = MAIN_COLS + wid
        cps = [
            pltpu.async_copy(
                table_t.at[pl.ds(8 * i, 8), pl.ds(j * 128, 128)],
                buf_a.at[pl.ds(8 * i, 8), pl.ds(0, 128)], in_sem_a)
            for i in range(4)
        ]
        for cp in cps:
            cp.wait()
        transpose_col(buf_a, 0, 0)
        pltpu.sync_copy(row_buf.at[pl.ds(0, COL_FLOATS)],
                        out_hbm.at[pl.ds(j * COL_FLOATS, COL_FLOATS)])

    @pl.when(wid == NW - 1)
    def _tail():
        cps = [
            pltpu.async_copy(tail_pad.at[pl.ds(8 * i, 8)],
                             buf_a.at[pl.ds(8 * i, 8), pl.ds(0, 128)],
                             in_sem_a)
            for i in range(4)
        ]
        for cp in cps:
            cp.wait()
        for g in range(TAIL_ROWS // L):
            vs = [buf_a[c, pl.ds(g * L, L)] for c in range(EMBED_DIM)]
            for c in range(EMBED_DIM):
                plsc.store_scatter(
                    row_buf, [iota_p + (g * L * PITCH + c)], vs[c])
        pltpu.sync_copy(
            row_buf.at[pl.ds(0, TAIL_ROWS * PITCH)],
            out_hbm.at[pl.ds(TCOLS * COL_FLOATS, TAIL_ROWS * PITCH)])


@functools.partial(
    pl.kernel,
    out_type=jax.ShapeDtypeStruct((EMBED_DIM, BATCH), jnp.float32),
    mesh=plsc.VectorSubcoreMesh(core_axis_name="c", subcore_axis_name="s"),
    compiler_params=pltpu.CompilerParams(
        needs_layout_passes=False, use_tc_tiling_on_sc=False),
    scratch_types=[
        pltpu.VMEM((B_PER_W,), jnp.int32),
        pltpu.VMEM((B_PER_W, PITCH), jnp.float32),
        pltpu.VMEM((EMBED_DIM, B_PER_W), jnp.float32),
        pltpu.VMEM((EMBED_DIM,), jnp.float32),
        pltpu.VMEM((EMBED_DIM,), jnp.float32),
        pltpu.SemaphoreType.DMA,
        pltpu.SemaphoreType.DMA,
        pltpu.SemaphoreType.DMA,
        pltpu.SemaphoreType.DMA,
    ],
)
def _sc_embed_ln(idx_hbm, table_hbm, gamma_hbm, beta_hbm, out_hbm,
                 idx_v, rows_v, rows_t, gamma_v, beta_v, s0, s1, s2, s3):
    wid = lax.axis_index("s") * NC + lax.axis_index("c")
    base = wid * B_PER_W

    pre = [pltpu.async_copy(gamma_hbm, gamma_v, s0),
           pltpu.async_copy(beta_hbm, beta_v, s0),
           pltpu.async_copy(idx_hbm.at[pl.ds(base, B_PER_W)], idx_v, s0)]
    for cp in pre:
        cp.wait()

    sems = [s0, s1, s2, s3]
    copies = []
    for c in range(N_CHUNKS):
        sl = pl.ds(c * GATHER_CHUNK, GATHER_CHUNK)
        copies.append(
            pltpu.async_copy(table_hbm.at[idx_v.at[sl]], rows_v.at[sl],
                             sems[c]))

    iota16 = lax.iota(jnp.int32, L)
    inv_d = jnp.float32(1.0 / EMBED_DIM)
    g_lo = gamma_v[pl.ds(0, L)]
    g_hi = gamma_v[pl.ds(L, L)]
    b_lo = beta_v[pl.ds(0, L)]
    b_hi = beta_v[pl.ds(L, L)]

    def group_body(g, carry):
        ridx = iota16 + g * L
        cols = []
        acc = [None] * 4
        for j in range(EMBED_DIM):
            cj = jnp.full((L,), j, jnp.int32)
            v = plsc.load_gather(rows_v, [ridx, cj])
            cols.append(v)
            k = j & 3
            acc[k] = v if acc[k] is None else acc[k] + v
        mean = ((acc[0] + acc[1]) + (acc[2] + acc[3])) * inv_d
        qcc = [None] * 4
        for j in range(EMBED_DIM):
            cols[j] = cols[j] - mean
            sq = cols[j] * cols[j]
            k = j & 3
            qcc[k] = sq if qcc[k] is None else qcc[k] + sq
        q = (qcc[0] + qcc[1]) + (qcc[2] + qcc[3])
        scale = _rsqrt(q * inv_d + jnp.float32(EPS))
        for j in range(EMBED_DIM):
            gj = _lane_broadcast(g_lo if j < L else g_hi, j % L)
            bj = _lane_broadcast(b_lo if j < L else b_hi, j % L)
            o = cols[j] * (scale * gj) + bj
            rows_t[j, pl.ds(g * L, L)] = o
        return carry

    for c in range(N_CHUNKS):
        copies[c].wait()
        lax.fori_loop(c * GROUPS_PER_CHUNK, (c + 1) * GROUPS_PER_CHUNK,
                      group_body, 0)

    pltpu.sync_copy(rows_t, out_hbm.at[:, pl.ds(base, B_PER_W)])


def kernel(action_indices, table, gamma, beta):
    tail_pad = jnp.pad(table[TCOLS * 128:].T, ((0, 0), (0, 128 - TAIL_ROWS)))
    flat = _sc_detile(table.T, tail_pad)
    out_t = _sc_embed_ln(action_indices.astype(jnp.int32),
                         flat.reshape(ROWS_PAD, PITCH), gamma, beta)
    return out_t.T


# speedup vs baseline: 3.5446x; 3.5446x over previous
import functools

import jax
import jax.numpy as jnp
from jax import lax
from jax.experimental import pallas as pl
from jax.experimental.pallas import tpu as pltpu
from jax.experimental.pallas import tpu_sc as plsc

NUM_ACTIONS = 100000
EMBED_DIM = 32
BATCH = 16384
EPS = 1e-5

NC = 2
NS = 16
L = 16
NW = NC * NS
B_PER_W = BATCH // NW
GATHER_CHUNK = 128
N_CHUNKS = B_PER_W // GATHER_CHUNK
GROUPS_PER_CHUNK = GATHER_CHUNK // L

TCOLS = NUM_ACTIONS // 128
TAIL_ROWS = NUM_ACTIONS - TCOLS * 128
ROWS_PAD = NUM_ACTIONS + 96
PITCH = EMBED_DIM + 8
FLAT_PAD = ROWS_PAD * PITCH

_GATHER_DNUMS = lax.GatherDimensionNumbers(
    offset_dims=(), collapsed_slice_dims=(0,), start_index_map=(0,))


def _lane_broadcast(vec, j):
    sel = jnp.full((L, 1), j, jnp.int32)
    return lax.gather(vec, sel, _GATHER_DNUMS, slice_sizes=(1,),
                      mode=lax.GatherScatterMode.PROMISE_IN_BOUNDS)


def _rsqrt(x):
    xi = plsc.bitcast(x, jnp.int32)
    y = plsc.bitcast(jnp.int32(0x5F3759DF) - (xi >> 1), jnp.float32)
    half = x * 0.5
    y = y * (1.5 - half * y * y)
    y = y * (1.5 - half * y * y)
    return y


M = 8
PASSES = 3
MAIN_COLS = NW * M * PASSES
EXTRA_COLS = TCOLS - MAIN_COLS
COL_FLOATS = 128 * PITCH


@functools.partial(
    pl.kernel,
    out_type=jax.ShapeDtypeStruct((FLAT_PAD,), jnp.float32),
    mesh=plsc.VectorSubcoreMesh(core_axis_name="c", subcore_axis_name="s"),
    compiler_params=pltpu.CompilerParams(
        needs_layout_passes=False, use_tc_tiling_on_sc=True),
    scratch_types=[
        pltpu.VMEM((EMBED_DIM, M * 128), jnp.float32),
        pltpu.VMEM((EMBED_DIM, M * 128), jnp.float32),
        pltpu.VMEM((M * COL_FLOATS,), jnp.float32),
        pltpu.SemaphoreType.DMA,
        pltpu.SemaphoreType.DMA,
        pltpu.SemaphoreType.DMA,
    ],
)
def _sc_detile(table_t, tail_pad, out_hbm, buf_a, buf_b, row_buf,
               in_sem_a, in_sem_b, out_sem):
    wid = lax.axis_index("s") * NC + lax.axis_index("c")
    j0 = wid * (M * PASSES)

    iota_p = lax.iota(jnp.int32, L) * PITCH
    bufs = [buf_a, buf_b]
    in_sems = [in_sem_a, in_sem_b]

    def fire_in(p, buf, sem):
        return [
            pltpu.async_copy(
                table_t.at[pl.ds(8 * i, 8),
                           pl.ds((j0 + M * p) * 128, M * 128)],
                buf.at[pl.ds(8 * i, 8)], sem)
            for i in range(4)
        ]

    def transpose_col(buf, c2, base):
        for g in range(8):
            vs = [buf[c, pl.ds(c2 * 128 + g * L, L)]
                  for c in range(EMBED_DIM)]
            for c in range(EMBED_DIM):
                plsc.store_scatter(
                    row_buf, [iota_p + (base + g * L * PITCH + c)],
                    vs[c])

    in_flight = fire_in(0, bufs[0], in_sems[0])
    out_copy = None
    for p in range(PASSES):
        for cp in in_flight:
            cp.wait()
        if p + 1 < PASSES:
            in_flight = fire_in(p + 1, bufs[(p + 1) % 2], in_sems[(p + 1) % 2])
        if out_copy is not None:
            out_copy.wait()
        buf = bufs[p % 2]

        def col_body(c2, carry, buf=buf):
            transpose_col(buf, c2, c2 * COL_FLOATS)
            return carry
        lax.fori_loop(0, M, col_body, 0)
        out_copy = pltpu.async_copy(
            row_buf, out_hbm.at[pl.ds((j0 + M * p) * COL_FLOATS,
                                      M * COL_FLOATS)], out_sem)
    out_copy.wait()

    @pl.when(wid < EXTRA_COLS)
    def _extra():
        j = MAIN_COLS + wid
        cps = [
            pltpu.async_copy(
                table_t.at[pl.ds(8 * i, 8), pl.ds(j * 128, 128)],
                buf_a.at[pl.ds(8 * i, 8), pl.ds(0, 128)], in_sem_a)
            for i in range(4)
        ]
        for cp in cps:
            cp.wait()
        transpose_col(buf_a, 0, 0)
        pltpu.sync_copy(row_buf.at[pl.ds(0, COL_FLOATS)],
                        out_hbm.at[pl.ds(j * COL_FLOATS, COL_FLOATS)])

    @pl.when(wid == NW - 1)
    def _tail():
        cps = [
            pltpu.async_copy(tail_pad.at[pl.ds(8 * i, 8)],
                             buf_a.at[pl.ds(8 * i, 8), pl.ds(0, 128)],
                             in_sem_a)
            for i in range(4)
        ]
        for cp in cps:
            cp.wait()
        for g in range(TAIL_ROWS // L):
            vs = [buf_a[c, pl.ds(g * L, L)] for c in range(EMBED_DIM)]
            for c in range(EMBED_DIM):
                plsc.store_scatter(
                    row_buf, [iota_p + (g * L * PITCH + c)], vs[c])
        pltpu.sync_copy(
            row_buf.at[pl.ds(0, TAIL_ROWS * PITCH)],
            out_hbm.at[pl.ds(TCOLS * COL_FLOATS, TAIL_ROWS * PITCH)])


@functools.partial(
    pl.kernel,
    out_type=jax.ShapeDtypeStruct((EMBED_DIM, BATCH), jnp.float32),
    mesh=plsc.VectorSubcoreMesh(core_axis_name="c", subcore_axis_name="s"),
    compiler_params=pltpu.CompilerParams(
        needs_layout_passes=False, use_tc_tiling_on_sc=False),
    scratch_types=[
        pltpu.VMEM((B_PER_W,), jnp.int32),
        pltpu.VMEM((B_PER_W, PITCH), jnp.float32),
        pltpu.VMEM((EMBED_DIM, B_PER_W), jnp.float32),
        pltpu.VMEM((EMBED_DIM,), jnp.float32),
        pltpu.VMEM((EMBED_DIM,), jnp.float32),
        pltpu.SemaphoreType.DMA,
        pltpu.SemaphoreType.DMA,
        pltpu.SemaphoreType.DMA,
        pltpu.SemaphoreType.DMA,
    ],
)
def _sc_embed_ln(idx_hbm, table_hbm, gamma_hbm, beta_hbm, out_hbm,
                 idx_v, rows_v, rows_t, gamma_v, beta_v, s0, s1, s2, s3):
    wid = lax.axis_index("s") * NC + lax.axis_index("c")
    base = wid * B_PER_W

    pre = [pltpu.async_copy(gamma_hbm, gamma_v, s0),
           pltpu.async_copy(beta_hbm, beta_v, s0),
           pltpu.async_copy(idx_hbm.at[pl.ds(base, B_PER_W)], idx_v, s0)]
    for cp in pre:
        cp.wait()

    sems = [s0, s1, s2, s3]
    copies = []
    for c in range(N_CHUNKS):
        sl = pl.ds(c * GATHER_CHUNK, GATHER_CHUNK)
        copies.append(
            pltpu.async_copy(table_hbm.at[idx_v.at[sl]], rows_v.at[sl],
                             sems[c]))

    iota16 = lax.iota(jnp.int32, L)
    inv_d = jnp.float32(1.0 / EMBED_DIM)
    g_lo = gamma_v[pl.ds(0, L)]
    g_hi = gamma_v[pl.ds(L, L)]
    b_lo = beta_v[pl.ds(0, L)]
    b_hi = beta_v[pl.ds(L, L)]

    def group_body(g, carry):
        ridx = iota16 + g * L
        cols = []
        acc = [None] * 4
        for j in range(EMBED_DIM):
            cj = jnp.full((L,), j, jnp.int32)
            v = plsc.load_gather(rows_v, [ridx, cj])
            cols.append(v)
            k = j & 3
            acc[k] = v if acc[k] is None else acc[k] + v
        mean = ((acc[0] + acc[1]) + (acc[2] + acc[3])) * inv_d
        qcc = [None] * 4
        for j in range(EMBED_DIM):
            cols[j] = cols[j] - mean
            sq = cols[j] * cols[j]
            k = j & 3
            qcc[k] = sq if qcc[k] is None else qcc[k] + sq
        q = (qcc[0] + qcc[1]) + (qcc[2] + qcc[3])
        scale = _rsqrt(q * inv_d + jnp.float32(EPS))
        for j in range(EMBED_DIM):
            gj = _lane_broadcast(g_lo if j < L else g_hi, j % L)
            bj = _lane_broadcast(b_lo if j < L else b_hi, j % L)
            o = cols[j] * (scale * gj) + bj
            rows_t[j, pl.ds(g * L, L)] = o
        return carry

    for c in range(N_CHUNKS):
        copies[c].wait()
        lax.fori_loop(c * GROUPS_PER_CHUNK, (c + 1) * GROUPS_PER_CHUNK,
                      group_body, 0)

    pltpu.sync_copy(rows_t, out_hbm.at[:, pl.ds(base, B_PER_W)])


def kernel(action_indices, table, gamma, beta):
    tail_pad = jnp.pad(table[TCOLS * 128:].T, ((0, 0), (0, 128 - TAIL_ROWS)))
    flat = _sc_detile(table.T, tail_pad)
    out_t = _sc_embed_ln(action_indices.astype(jnp.int32),
                         flat.reshape(ROWS_PAD, PITCH), gamma, beta)
    return out_t.T


# speedup vs baseline: 3.6911x; 1.0413x over previous
import functools

import jax
import jax.numpy as jnp
from jax import lax
from jax.experimental import pallas as pl
from jax.experimental.pallas import tpu as pltpu
from jax.experimental.pallas import tpu_sc as plsc

NUM_ACTIONS = 100000
EMBED_DIM = 32
BATCH = 16384
EPS = 1e-5

NC = 2
NS = 16
L = 16
NW = NC * NS
B_PER_W = BATCH // NW
GATHER_CHUNK = 128
N_CHUNKS = B_PER_W // GATHER_CHUNK
GROUPS_PER_CHUNK = GATHER_CHUNK // L

TCOLS = NUM_ACTIONS // 128
TAIL_ROWS = NUM_ACTIONS - TCOLS * 128
ROWS_PAD = NUM_ACTIONS + 96
PITCH = EMBED_DIM + 8
FLAT_PAD = ROWS_PAD * PITCH

_GATHER_DNUMS = lax.GatherDimensionNumbers(
    offset_dims=(), collapsed_slice_dims=(0,), start_index_map=(0,))


def _lane_broadcast(vec, j):
    sel = jnp.full((L, 1), j, jnp.int32)
    return lax.gather(vec, sel, _GATHER_DNUMS, slice_sizes=(1,),
                      mode=lax.GatherScatterMode.PROMISE_IN_BOUNDS)


def _rsqrt(x):
    xi = plsc.bitcast(x, jnp.int32)
    y = plsc.bitcast(jnp.int32(0x5F3759DF) - (xi >> 1), jnp.float32)
    half = x * 0.5
    y = y * (1.5 - half * y * y)
    y = y * (1.5 - half * y * y)
    return y


M = 8
PASSES = 3
MAIN_COLS = NW * M * PASSES
EXTRA_COLS = TCOLS - MAIN_COLS
COL_FLOATS = 128 * PITCH


@functools.partial(
    pl.kernel,
    out_type=jax.ShapeDtypeStruct((FLAT_PAD,), jnp.float32),
    mesh=plsc.VectorSubcoreMesh(core_axis_name="c", subcore_axis_name="s"),
    compiler_params=pltpu.CompilerParams(
        needs_layout_passes=False, use_tc_tiling_on_sc=True),
    scratch_types=[
        pltpu.VMEM((EMBED_DIM, M * 128), jnp.float32),
        pltpu.VMEM((EMBED_DIM, M * 128), jnp.float32),
        pltpu.VMEM((M * COL_FLOATS,), jnp.float32),
        pltpu.SemaphoreType.DMA,
        pltpu.SemaphoreType.DMA,
        pltpu.SemaphoreType.DMA,
    ],
)
def _sc_detile(table_t, tail_pad, out_hbm, buf_a, buf_b, row_buf,
               in_sem_a, in_sem_b, out_sem):
    wid = lax.axis_index("s") * NC + lax.axis_index("c")
    j0 = wid * (M * PASSES)

    iota_p = lax.iota(jnp.int32, L) * PITCH
    bufs = [buf_a, buf_b]
    in_sems = [in_sem_a, in_sem_b]

    def fire_in(p, buf, sem):
        return [
            pltpu.async_copy(
                table_t.at[pl.ds(8 * i, 8),
                           pl.ds((j0 + M * p) * 128, M * 128)],
                buf.at[pl.ds(8 * i, 8)], sem)
            for i in range(4)
        ]

    def transpose_col(buf, c2, base):
        for g in range(8):
            vs = [buf[c, pl.ds(c2 * 128 + g * L, L)]
                  for c in range(EMBED_DIM)]
            for c in range(EMBED_DIM):
                plsc.store_scatter(
                    row_buf, [iota_p + (base + g * L * PITCH + c)],
                    vs[c])

    in_flight = fire_in(0, bufs[0], in_sems[0])
    out_copy = None
    for p in range(PASSES):
        for cp in in_flight:
            cp.wait()
        if p + 1 < PASSES:
            in_flight = fire_in(p + 1, bufs[(p + 1) % 2], in_sems[(p + 1) % 2])
        if out_copy is not None:
            out_copy.wait()
        buf = bufs[p % 2]

        def grp_body(i, carry, buf=buf):
            c2 = i // 8
            g = i - c2 * 8
            vs = [buf[c, pl.ds(c2 * 128 + g * L, L)]
                  for c in range(EMBED_DIM)]
            base = c2 * COL_FLOATS + g * (L * PITCH)
            for c in range(EMBED_DIM):
                plsc.store_scatter(row_buf, [iota_p + (base + c)], vs[c])
            return carry
        lax.fori_loop(0, M * 8, grp_body, 0)
        out_copy = pltpu.async_copy(
            row_buf, out_hbm.at[pl.ds((j0 + M * p) * COL_FLOATS,
                                      M * COL_FLOATS)], out_sem)
    out_copy.wait()

    @pl.when(wid < EXTRA_COLS)
    def _extra():
        j = MAIN_COLS + wid
        cps = [
            pltpu.async_copy(
                table_t.at[pl.ds(8 * i, 8), pl.ds(j * 128, 128)],
                buf_a.at[pl.ds(8 * i, 8), pl.ds(0, 128)], in_sem_a)
            for i in range(4)
        ]
        for cp in cps:
            cp.wait()
        transpose_col(buf_a, 0, 0)
        pltpu.sync_copy(row_buf.at[pl.ds(0, COL_FLOATS)],
                        out_hbm.at[pl.ds(j * COL_FLOATS, COL_FLOATS)])

    @pl.when(wid == NW - 1)
    def _tail():
        cps = [
            pltpu.async_copy(tail_pad.at[pl.ds(8 * i, 8)],
                             buf_a.at[pl.ds(8 * i, 8), pl.ds(0, 128)],
                             in_sem_a)
            for i in range(4)
        ]
        for cp in cps:
            cp.wait()
        for g in range(TAIL_ROWS // L):
            vs = [buf_a[c, pl.ds(g * L, L)] for c in range(EMBED_DIM)]
            for c in range(EMBED_DIM):
                plsc.store_scatter(
                    row_buf, [iota_p + (g * L * PITCH + c)], vs[c])
        pltpu.sync_copy(
            row_buf.at[pl.ds(0, TAIL_ROWS * PITCH)],
            out_hbm.at[pl.ds(TCOLS * COL_FLOATS, TAIL_ROWS * PITCH)])


@functools.partial(
    pl.kernel,
    out_type=jax.ShapeDtypeStruct((EMBED_DIM, BATCH), jnp.float32),
    mesh=plsc.VectorSubcoreMesh(core_axis_name="c", subcore_axis_name="s"),
    compiler_params=pltpu.CompilerParams(
        needs_layout_passes=False, use_tc_tiling_on_sc=False),
    scratch_types=[
        pltpu.VMEM((B_PER_W,), jnp.int32),
        pltpu.VMEM((B_PER_W, PITCH), jnp.float32),
        pltpu.VMEM((EMBED_DIM, B_PER_W), jnp.float32),
        pltpu.VMEM((EMBED_DIM,), jnp.float32),
        pltpu.VMEM((EMBED_DIM,), jnp.float32),
        pltpu.SemaphoreType.DMA,
        pltpu.SemaphoreType.DMA,
        pltpu.SemaphoreType.DMA,
        pltpu.SemaphoreType.DMA,
    ],
)
def _sc_embed_ln(idx_hbm, table_hbm, gamma_hbm, beta_hbm, out_hbm,
                 idx_v, rows_v, rows_t, gamma_v, beta_v, s0, s1, s2, s3):
    wid = lax.axis_index("s") * NC + lax.axis_index("c")
    base = wid * B_PER_W

    pre = [pltpu.async_copy(gamma_hbm, gamma_v, s0),
           pltpu.async_copy(beta_hbm, beta_v, s0),
           pltpu.async_copy(idx_hbm.at[pl.ds(base, B_PER_W)], idx_v, s0)]
    for cp in pre:
        cp.wait()

    sems = [s0, s1, s2, s3]
    copies = []
    for c in range(N_CHUNKS):
        sl = pl.ds(c * GATHER_CHUNK, GATHER_CHUNK)
        copies.append(
            pltpu.async_copy(table_hbm.at[idx_v.at[sl]], rows_v.at[sl],
                             sems[c]))

    iota16 = lax.iota(jnp.int32, L)
    inv_d = jnp.float32(1.0 / EMBED_DIM)
    g_lo = gamma_v[pl.ds(0, L)]
    g_hi = gamma_v[pl.ds(L, L)]
    b_lo = beta_v[pl.ds(0, L)]
    b_hi = beta_v[pl.ds(L, L)]

    def group_body(g, carry):
        ridx = iota16 + g * L
        cols = []
        acc = [None] * 4
        for j in range(EMBED_DIM):
            cj = jnp.full((L,), j, jnp.int32)
            v = plsc.load_gather(rows_v, [ridx, cj])
            cols.append(v)
            k = j & 3
            acc[k] = v if acc[k] is None else acc[k] + v
        mean = ((acc[0] + acc[1]) + (acc[2] + acc[3])) * inv_d
        qcc = [None] * 4
        for j in range(EMBED_DIM):
            cols[j] = cols[j] - mean
            sq = cols[j] * cols[j]
            k = j & 3
            qcc[k] = sq if qcc[k] is None else qcc[k] + sq
        q = (qcc[0] + qcc[1]) + (qcc[2] + qcc[3])
        scale = _rsqrt(q * inv_d + jnp.float32(EPS))
        for j in range(EMBED_DIM):
            gj = _lane_broadcast(g_lo if j < L else g_hi, j % L)
            bj = _lane_broadcast(b_lo if j < L else b_hi, j % L)
            o = cols[j] * (scale * gj) + bj
            rows_t[j, pl.ds(g * L, L)] = o
        return carry

    for c in range(N_CHUNKS):
        copies[c].wait()
        lax.fori_loop(c * GROUPS_PER_CHUNK, (c + 1) * GROUPS_PER_CHUNK,
                      group_body, 0)

    pltpu.sync_copy(rows_t, out_hbm.at[:, pl.ds(base, B_PER_W)])


def kernel(action_indices, table, gamma, beta):
    tail_pad = jnp.pad(table[TCOLS * 128:].T, ((0, 0), (0, 128 - TAIL_ROWS)))
    flat = _sc_detile(table.T, tail_pad)
    out_t = _sc_embed_ln(action_indices.astype(jnp.int32),
                         flat.reshape(ROWS_PAD, PITCH), gamma, beta)
    return out_t.T


# speedup vs baseline: 3.7161x; 1.0068x over previous
import functools

import jax
import jax.numpy as jnp
from jax import lax
from jax.experimental import pallas as pl
from jax.experimental.pallas import tpu as pltpu
from jax.experimental.pallas import tpu_sc as plsc

NUM_ACTIONS = 100000
EMBED_DIM = 32
BATCH = 16384
EPS = 1e-5

NC = 2
NS = 16
L = 16
NW = NC * NS
B_PER_W = BATCH // NW
GATHER_CHUNK = 128
N_CHUNKS = B_PER_W // GATHER_CHUNK
GROUPS_PER_CHUNK = GATHER_CHUNK // L

TCOLS = NUM_ACTIONS // 128
TAIL_ROWS = NUM_ACTIONS - TCOLS * 128
ROWS_PAD = NUM_ACTIONS + 96
PITCH = EMBED_DIM + 8
FLAT_PAD = ROWS_PAD * PITCH

_GATHER_DNUMS = lax.GatherDimensionNumbers(
    offset_dims=(), collapsed_slice_dims=(0,), start_index_map=(0,))


def _lane_broadcast(vec, j):
    sel = jnp.full((L, 1), j, jnp.int32)
    return lax.gather(vec, sel, _GATHER_DNUMS, slice_sizes=(1,),
                      mode=lax.GatherScatterMode.PROMISE_IN_BOUNDS)


def _rsqrt(x):
    xi = plsc.bitcast(x, jnp.int32)
    y = plsc.bitcast(jnp.int32(0x5F3759DF) - (xi >> 1), jnp.float32)
    half = x * 0.5
    y = y * (1.5 - half * y * y)
    y = y * (1.5 - half * y * y)
    return y


M = 8
PASSES = 3
MAIN_COLS = NW * M * PASSES
EXTRA_COLS = TCOLS - MAIN_COLS
COL_FLOATS = 128 * PITCH


@functools.partial(
    pl.kernel,
    out_type=jax.ShapeDtypeStruct((FLAT_PAD,), jnp.float32),
    mesh=plsc.VectorSubcoreMesh(core_axis_name="c", subcore_axis_name="s"),
    compiler_params=pltpu.CompilerParams(
        needs_layout_passes=False, use_tc_tiling_on_sc=True),
    scratch_types=[
        pltpu.VMEM((EMBED_DIM, M * 128), jnp.float32),
        pltpu.VMEM((EMBED_DIM, M * 128), jnp.float32),
        pltpu.VMEM((M * COL_FLOATS,), jnp.float32),
        pltpu.SemaphoreType.DMA,
        pltpu.SemaphoreType.DMA,
        pltpu.SemaphoreType.DMA,
    ],
)
def _sc_detile(table_t, tail_pad, out_hbm, buf_a, buf_b, row_buf,
               in_sem_a, in_sem_b, out_sem):
    wid = lax.axis_index("s") * NC + lax.axis_index("c")
    j0 = wid * (M * PASSES)

    iota_p = lax.iota(jnp.int32, L) * PITCH
    bufs = [buf_a, buf_b]
    in_sems = [in_sem_a, in_sem_b]

    def fire_in(p, buf, sem):
        return [
            pltpu.async_copy(
                table_t.at[pl.ds(8 * i, 8),
                           pl.ds((j0 + M * p) * 128, M * 128)],
                buf.at[pl.ds(8 * i, 8)], sem)
            for i in range(4)
        ]

    def transpose_col(buf, c2, base):
        for g in range(8):
            vs = [buf[c, pl.ds(c2 * 128 + g * L, L)]
                  for c in range(EMBED_DIM)]
            for c in range(EMBED_DIM):
                plsc.store_scatter(
                    row_buf, [iota_p + (base + g * L * PITCH + c)],
                    vs[c])

    in_flight = fire_in(0, bufs[0], in_sems[0])
    out_copy = None
    for p in range(PASSES):
        for cp in in_flight:
            cp.wait()
        if p + 1 < PASSES:
            in_flight = fire_in(p + 1, bufs[(p + 1) % 2], in_sems[(p + 1) % 2])
        if out_copy is not None:
            out_copy.wait()
        buf = bufs[p % 2]

        def grp_body(i, carry, buf=buf):
            c2 = i // 4
            g2 = (i - c2 * 4) * 2
            for g in (g2, g2 + 1):
                vs = [buf[c, pl.ds(c2 * 128 + g * L, L)]
                      for c in range(EMBED_DIM)]
                base = c2 * COL_FLOATS + g * (L * PITCH)
                for c in range(EMBED_DIM):
                    plsc.store_scatter(row_buf, [iota_p + (base + c)],
                                       vs[c])
            return carry
        lax.fori_loop(0, M * 4, grp_body, 0)
        out_copy = pltpu.async_copy(
            row_buf, out_hbm.at[pl.ds((j0 + M * p) * COL_FLOATS,
                                      M * COL_FLOATS)], out_sem)
    out_copy.wait()

    @pl.when(wid < EXTRA_COLS)
    def _extra():
        j = MAIN_COLS + wid
        cps = [
            pltpu.async_copy(
                table_t.at[pl.ds(8 * i, 8), pl.ds(j * 128, 128)],
                buf_a.at[pl.ds(8 * i, 8), pl.ds(0, 128)], in_sem_a)
            for i in range(4)
        ]
        for cp in cps:
            cp.wait()
        transpose_col(buf_a, 0, 0)
        pltpu.sync_copy(row_buf.at[pl.ds(0, COL_FLOATS)],
                        out_hbm.at[pl.ds(j * COL_FLOATS, COL_FLOATS)])

    @pl.when(wid == NW - 1)
    def _tail():
        cps = [
            pltpu.async_copy(tail_pad.at[pl.ds(8 * i, 8)],
                             buf_a.at[pl.ds(8 * i, 8), pl.ds(0, 128)],
                             in_sem_a)
            for i in range(4)
        ]
        for cp in cps:
            cp.wait()
        for g in range(TAIL_ROWS // L):
            vs = [buf_a[c, pl.ds(g * L, L)] for c in range(EMBED_DIM)]
            for c in range(EMBED_DIM):
                plsc.store_scatter(
                    row_buf, [iota_p + (g * L * PITCH + c)], vs[c])
        pltpu.sync_copy(
            row_buf.at[pl.ds(0, TAIL_ROWS * PITCH)],
            out_hbm.at[pl.ds(TCOLS * COL_FLOATS, TAIL_ROWS * PITCH)])


@functools.partial(
    pl.kernel,
    out_type=jax.ShapeDtypeStruct((EMBED_DIM, BATCH), jnp.float32),
    mesh=plsc.VectorSubcoreMesh(core_axis_name="c", subcore_axis_name="s"),
    compiler_params=pltpu.CompilerParams(
        needs_layout_passes=False, use_tc_tiling_on_sc=False),
    scratch_types=[
        pltpu.VMEM((B_PER_W,), jnp.int32),
        pltpu.VMEM((B_PER_W, PITCH), jnp.float32),
        pltpu.VMEM((EMBED_DIM, B_PER_W), jnp.float32),
        pltpu.VMEM((EMBED_DIM,), jnp.float32),
        pltpu.VMEM((EMBED_DIM,), jnp.float32),
        pltpu.SemaphoreType.DMA,
        pltpu.SemaphoreType.DMA,
        pltpu.SemaphoreType.DMA,
        pltpu.SemaphoreType.DMA,
    ],
)
def _sc_embed_ln(idx_hbm, table_hbm, gamma_hbm, beta_hbm, out_hbm,
                 idx_v, rows_v, rows_t, gamma_v, beta_v, s0, s1, s2, s3):
    wid = lax.axis_index("s") * NC + lax.axis_index("c")
    base = wid * B_PER_W

    pre = [pltpu.async_copy(gamma_hbm, gamma_v, s0),
           pltpu.async_copy(beta_hbm, beta_v, s0),
           pltpu.async_copy(idx_hbm.at[pl.ds(base, B_PER_W)], idx_v, s0)]
    for cp in pre:
        cp.wait()

    sems = [s0, s1, s2, s3]
    copies = []
    for c in range(N_CHUNKS):
        sl = pl.ds(c * GATHER_CHUNK, GATHER_CHUNK)
        copies.append(
            pltpu.async_copy(table_hbm.at[idx_v.at[sl]], rows_v.at[sl],
                             sems[c]))

    iota16 = lax.iota(jnp.int32, L)
    inv_d = jnp.float32(1.0 / EMBED_DIM)
    g_lo = gamma_v[pl.ds(0, L)]
    g_hi = gamma_v[pl.ds(L, L)]
    b_lo = beta_v[pl.ds(0, L)]
    b_hi = beta_v[pl.ds(L, L)]

    def group_body(g, carry):
        ridx = iota16 + g * L
        cols = []
        acc = [None] * 4
        for j in range(EMBED_DIM):
            cj = jnp.full((L,), j, jnp.int32)
            v = plsc.load_gather(rows_v, [ridx, cj])
            cols.append(v)
            k = j & 3
            acc[k] = v if acc[k] is None else acc[k] + v
        mean = ((acc[0] + acc[1]) + (acc[2] + acc[3])) * inv_d
        qcc = [None] * 4
        for j in range(EMBED_DIM):
            cols[j] = cols[j] - mean
            sq = cols[j] * cols[j]
            k = j & 3
            qcc[k] = sq if qcc[k] is None else qcc[k] + sq
        q = (qcc[0] + qcc[1]) + (qcc[2] + qcc[3])
        scale = _rsqrt(q * inv_d + jnp.float32(EPS))
        for j in range(EMBED_DIM):
            gj = _lane_broadcast(g_lo if j < L else g_hi, j % L)
            bj = _lane_broadcast(b_lo if j < L else b_hi, j % L)
            o = cols[j] * (scale * gj) + bj
            rows_t[j, pl.ds(g * L, L)] = o
        return carry

    for c in range(N_CHUNKS):
        copies[c].wait()
        lax.fori_loop(c * GROUPS_PER_CHUNK, (c + 1) * GROUPS_PER_CHUNK,
                      group_body, 0)

    pltpu.sync_copy(rows_t, out_hbm.at[:, pl.ds(base, B_PER_W)])


def kernel(action_indices, table, gamma, beta):
    tail_pad = jnp.pad(table[TCOLS * 128:].T, ((0, 0), (0, 128 - TAIL_ROWS)))
    flat = _sc_detile(table.T, tail_pad)
    out_t = _sc_embed_ln(action_indices.astype(jnp.int32),
                         flat.reshape(ROWS_PAD, PITCH), gamma, beta)
    return out_t.T


# speedup vs baseline: 3.7275x; 1.0031x over previous
import functools

import jax
import jax.numpy as jnp
from jax import lax
from jax.experimental import pallas as pl
from jax.experimental.pallas import tpu as pltpu
from jax.experimental.pallas import tpu_sc as plsc

NUM_ACTIONS = 100000
EMBED_DIM = 32
BATCH = 16384
EPS = 1e-5

NC = 2
NS = 16
L = 16
NW = NC * NS
B_PER_W = BATCH // NW
GATHER_CHUNK = 128
N_CHUNKS = B_PER_W // GATHER_CHUNK
GROUPS_PER_CHUNK = GATHER_CHUNK // L

TCOLS = NUM_ACTIONS // 128
TAIL_ROWS = NUM_ACTIONS - TCOLS * 128
ROWS_PAD = NUM_ACTIONS + 96
PITCH = EMBED_DIM + 8
FLAT_PAD = ROWS_PAD * PITCH

_GATHER_DNUMS = lax.GatherDimensionNumbers(
    offset_dims=(), collapsed_slice_dims=(0,), start_index_map=(0,))


def _lane_broadcast(vec, j):
    sel = jnp.full((L, 1), j, jnp.int32)
    return lax.gather(vec, sel, _GATHER_DNUMS, slice_sizes=(1,),
                      mode=lax.GatherScatterMode.PROMISE_IN_BOUNDS)


def _rsqrt(x):
    xi = plsc.bitcast(x, jnp.int32)
    y = plsc.bitcast(jnp.int32(0x5F3759DF) - (xi >> 1), jnp.float32)
    half = x * 0.5
    y = y * (1.5 - half * y * y)
    y = y * (1.5 - half * y * y)
    return y


M = 8
PASSES = 3
MAIN_COLS = NW * M * PASSES
EXTRA_COLS = TCOLS - MAIN_COLS
COL_FLOATS = 128 * PITCH


@functools.partial(
    pl.kernel,
    out_type=jax.ShapeDtypeStruct((FLAT_PAD,), jnp.float32),
    mesh=plsc.VectorSubcoreMesh(core_axis_name="c", subcore_axis_name="s"),
    compiler_params=pltpu.CompilerParams(
        needs_layout_passes=False, use_tc_tiling_on_sc=True),
    scratch_types=[
        pltpu.VMEM((EMBED_DIM, M * 128), jnp.float32),
        pltpu.VMEM((EMBED_DIM, M * 128), jnp.float32),
        pltpu.VMEM((M * COL_FLOATS,), jnp.float32),
        pltpu.SemaphoreType.DMA,
        pltpu.SemaphoreType.DMA,
        pltpu.SemaphoreType.DMA,
    ],
)
def _sc_detile(table_t, tail_pad, out_hbm, buf_a, buf_b, row_buf,
               in_sem_a, in_sem_b, out_sem):
    wid = lax.axis_index("s") * NC + lax.axis_index("c")
    j0 = wid * (M * PASSES)

    iota_p = lax.iota(jnp.int32, L) * PITCH
    bufs = [buf_a, buf_b]
    in_sems = [in_sem_a, in_sem_b]

    def fire_in(p, buf, sem):
        return [
            pltpu.async_copy(
                table_t.at[pl.ds(8 * i, 8),
                           pl.ds((j0 + M * p) * 128, M * 128)],
                buf.at[pl.ds(8 * i, 8)], sem)
            for i in range(4)
        ]

    def transpose_col(buf, c2, base):
        for g in range(8):
            vs = [buf[c, pl.ds(c2 * 128 + g * L, L)]
                  for c in range(EMBED_DIM)]
            for c in range(EMBED_DIM):
                plsc.store_scatter(
                    row_buf, [iota_p + (base + g * L * PITCH + c)],
                    vs[c])

    in_flight = fire_in(0, bufs[0], in_sems[0])
    out_copy = None
    for p in range(PASSES):
        for cp in in_flight:
            cp.wait()
        if p + 1 < PASSES:
            in_flight = fire_in(p + 1, bufs[(p + 1) % 2], in_sems[(p + 1) % 2])
        if out_copy is not None:
            out_copy.wait()
        buf = bufs[p % 2]

        def grp_body(i, carry, buf=buf):
            c2 = i // 4
            g2 = (i - c2 * 4) * 2
            for g in (g2, g2 + 1):
                vs = [buf[c, pl.ds(c2 * 128 + g * L, L)]
                      for c in range(EMBED_DIM)]
                base = c2 * COL_FLOATS + g * (L * PITCH)
                for c in range(EMBED_DIM):
                    plsc.store_scatter(row_buf, [iota_p + (base + c)],
                                       vs[c])
            return carry
        lax.fori_loop(0, M * 4, grp_body, 0)
        out_copy = pltpu.async_copy(
            row_buf, out_hbm.at[pl.ds((j0 + M * p) * COL_FLOATS,
                                      M * COL_FLOATS)], out_sem)
    out_copy.wait()

    @pl.when(wid < EXTRA_COLS)
    def _extra():
        j = MAIN_COLS + wid
        cps = [
            pltpu.async_copy(
                table_t.at[pl.ds(8 * i, 8), pl.ds(j * 128, 128)],
                buf_a.at[pl.ds(8 * i, 8), pl.ds(0, 128)], in_sem_a)
            for i in range(4)
        ]
        for cp in cps:
            cp.wait()
        transpose_col(buf_a, 0, 0)
        pltpu.sync_copy(row_buf.at[pl.ds(0, COL_FLOATS)],
                        out_hbm.at[pl.ds(j * COL_FLOATS, COL_FLOATS)])

    @pl.when(wid == NW - 1)
    def _tail():
        cps = [
            pltpu.async_copy(tail_pad.at[pl.ds(8 * i, 8)],
                             buf_a.at[pl.ds(8 * i, 8), pl.ds(0, 128)],
                             in_sem_a)
            for i in range(4)
        ]
        for cp in cps:
            cp.wait()
        for g in range(TAIL_ROWS // L):
            vs = [buf_a[c, pl.ds(g * L, L)] for c in range(EMBED_DIM)]
            for c in range(EMBED_DIM):
                plsc.store_scatter(
                    row_buf, [iota_p + (g * L * PITCH + c)], vs[c])
        pltpu.sync_copy(
            row_buf.at[pl.ds(0, TAIL_ROWS * PITCH)],
            out_hbm.at[pl.ds(TCOLS * COL_FLOATS, TAIL_ROWS * PITCH)])


@functools.partial(
    pl.kernel,
    out_type=jax.ShapeDtypeStruct((EMBED_DIM, BATCH), jnp.float32),
    mesh=plsc.VectorSubcoreMesh(core_axis_name="c", subcore_axis_name="s"),
    compiler_params=pltpu.CompilerParams(
        needs_layout_passes=False, use_tc_tiling_on_sc=False),
    scratch_types=[
        pltpu.VMEM((B_PER_W,), jnp.int32),
        pltpu.VMEM((B_PER_W, PITCH), jnp.float32),
        pltpu.VMEM((EMBED_DIM, B_PER_W), jnp.float32),
        pltpu.VMEM((EMBED_DIM,), jnp.float32),
        pltpu.VMEM((EMBED_DIM,), jnp.float32),
        pltpu.SemaphoreType.DMA,
        pltpu.SemaphoreType.DMA,
        pltpu.SemaphoreType.DMA,
        pltpu.SemaphoreType.DMA,
    ],
)
def _sc_embed_ln(idx_hbm, table_hbm, gamma_hbm, beta_hbm, out_hbm,
                 idx_v, rows_v, rows_t, gamma_v, beta_v, s0, s1, s2, s3):
    wid = lax.axis_index("s") * NC + lax.axis_index("c")
    base = wid * B_PER_W

    sems = [s0, s1, s2, s3]
    idx_copies = [
        pltpu.async_copy(
            idx_hbm.at[pl.ds(base + c * GATHER_CHUNK, GATHER_CHUNK)],
            idx_v.at[pl.ds(c * GATHER_CHUNK, GATHER_CHUNK)], sems[c])
        for c in range(N_CHUNKS)
    ]
    pre = [pltpu.async_copy(gamma_hbm, gamma_v, s0),
           pltpu.async_copy(beta_hbm, beta_v, s0)]
    copies = []
    for c in range(N_CHUNKS):
        idx_copies[c].wait()
        sl = pl.ds(c * GATHER_CHUNK, GATHER_CHUNK)
        copies.append(
            pltpu.async_copy(table_hbm.at[idx_v.at[sl]], rows_v.at[sl],
                             sems[c]))
    for cp in pre:
        cp.wait()

    iota16 = lax.iota(jnp.int32, L)
    inv_d = jnp.float32(1.0 / EMBED_DIM)
    g_lo = gamma_v[pl.ds(0, L)]
    g_hi = gamma_v[pl.ds(L, L)]
    b_lo = beta_v[pl.ds(0, L)]
    b_hi = beta_v[pl.ds(L, L)]

    def group_body(g, carry):
        ridx = iota16 + g * L
        cols = []
        acc = [None] * 4
        for j in range(EMBED_DIM):
            cj = jnp.full((L,), j, jnp.int32)
            v = plsc.load_gather(rows_v, [ridx, cj])
            cols.append(v)
            k = j & 3
            acc[k] = v if acc[k] is None else acc[k] + v
        mean = ((acc[0] + acc[1]) + (acc[2] + acc[3])) * inv_d
        qcc = [None] * 4
        for j in range(EMBED_DIM):
            cols[j] = cols[j] - mean
            sq = cols[j] * cols[j]
            k = j & 3
            qcc[k] = sq if qcc[k] is None else qcc[k] + sq
        q = (qcc[0] + qcc[1]) + (qcc[2] + qcc[3])
        scale = _rsqrt(q * inv_d + jnp.float32(EPS))
        for j in range(EMBED_DIM):
            gj = _lane_broadcast(g_lo if j < L else g_hi, j % L)
            bj = _lane_broadcast(b_lo if j < L else b_hi, j % L)
            o = cols[j] * (scale * gj) + bj
            rows_t[j, pl.ds(g * L, L)] = o
        return carry

    for c in range(N_CHUNKS):
        copies[c].wait()
        lax.fori_loop(c * GROUPS_PER_CHUNK, (c + 1) * GROUPS_PER_CHUNK,
                      group_body, 0)

    pltpu.sync_copy(rows_t, out_hbm.at[:, pl.ds(base, B_PER_W)])


def kernel(action_indices, table, gamma, beta):
    tail_pad = jnp.pad(table[TCOLS * 128:].T, ((0, 0), (0, 128 - TAIL_ROWS)))
    flat = _sc_detile(table.T, tail_pad)
    out_t = _sc_embed_ln(action_indices.astype(jnp.int32),
                         flat.reshape(ROWS_PAD, PITCH), gamma, beta)
    return out_t.T
